# baseline (speedup 1.0000x reference)
import functools

import jax
import jax.numpy as jnp
import numpy as np
from jax.experimental import pallas as pl
from jax.experimental.pallas import tpu as pltpu

N_OBJ = 600
N_REL = 1024
IN_FEAT = 2048
D_MODEL = 1936
N_HEADS = 8
HEAD_DIM = 242
HEAD_PAD = 256
QKV_PAD = N_HEADS * HEAD_PAD
NUM_CLASSES = 37
D_FF = 2048
HW = 49
C_U = 1024
D_MID = 256

_F32 = jnp.float32
_BF = jnp.bfloat16


def _bd(t):
    return t.astype(_BF)


def _k1p_body(wu_ref, wvrf_ref, wm_ref, bias_ref, bvr_ref,
              m2_ref, m2s_ref, const_ref, *, bc):
    c = pl.program_id(0)
    wvrf = wvrf_ref[...]
    m2 = jnp.dot(_bd(wu_ref[...]), wvrf,
                 preferred_element_type=_F32)
    m2_ref[...] = _bd(m2).reshape(bc * HW, 512)

    @pl.when(c == 0)
    def _():
        m2s = jnp.dot(_bd(wm_ref[...]), wvrf,
                      preferred_element_type=_F32)
        m2s_ref[...] = _bd(m2s).reshape(2 * HW, 512)
        cb = jnp.dot(_bd(bias_ref[...]), wvrf,
                     preferred_element_type=_F32)
        const_ref[...] = (jnp.sum(cb.reshape(HW, 512), axis=0, keepdims=True)
                          + bvr_ref[...])


def _run_k1p(wu, wvrf, wm, bias_um, b_vr, bc=128):
    grid = C_U // bc
    return pl.pallas_call(
        functools.partial(_k1p_body, bc=bc),
        grid=(grid,),
        in_specs=[
            pl.BlockSpec((bc, D_MID), lambda c: (c, 0)),
            pl.BlockSpec((D_MID, HW * 512), lambda c: (0, 0)),
            pl.BlockSpec((2, D_MID), lambda c: (0, 0)),
            pl.BlockSpec((1, D_MID), lambda c: (0, 0)),
            pl.BlockSpec((1, 512), lambda c: (0, 0)),
        ],
        out_specs=[
            pl.BlockSpec((bc * HW, 512), lambda c: (c, 0)),
            pl.BlockSpec((2 * HW, 512), lambda c: (0, 0)),
            pl.BlockSpec((1, 512), lambda c: (0, 0)),
        ],
        out_shape=[
            jax.ShapeDtypeStruct((C_U * HW, 512), _BF),
            jax.ShapeDtypeStruct((2 * HW, 512), _BF),
            jax.ShapeDtypeStruct((1, 512), _F32),
        ],
    )(wu, wvrf, wm, bias_um, b_vr)


def _k1m_body(u_ref, s_ref, m2_ref, m2s_ref, const_ref, out_ref):
    c = pl.program_id(1)
    part = jnp.dot(_bd(u_ref[...]), m2_ref[...],
                   preferred_element_type=_F32)

    @pl.when(c == 0)
    def _():
        out_ref[...] = (part + const_ref[...]
                        + jnp.dot(_bd(s_ref[...]), m2s_ref[...],
                                  preferred_element_type=_F32))

    @pl.when(c > 0)
    def _():
        out_ref[...] += part


def _run_k1m(u_flat, s_flat, m2, m2s, const, bn=512, bc=128):
    kb = bc * HW
    grid = (N_REL // bn, C_U // bc)
    return pl.pallas_call(
        _k1m_body,
        grid=grid,
        in_specs=[
            pl.BlockSpec((bn, kb), lambda n, c: (n, c)),
            pl.BlockSpec((bn, 2 * HW), lambda n, c: (n, 0)),
            pl.BlockSpec((kb, 512), lambda n, c: (c, 0)),
            pl.BlockSpec((2 * HW, 512), lambda n, c: (0, 0)),
            pl.BlockSpec((1, 512), lambda n, c: (0, 0)),
        ],
        out_specs=pl.BlockSpec((bn, 512), lambda n, c: (n, 0)),
        out_shape=jax.ShapeDtypeStruct((N_REL, 512), _F32),
        compiler_params=pltpu.CompilerParams(
            dimension_semantics=("parallel", "arbitrary")),
    )(u_flat, s_flat, m2, m2s, const)


def _k0_body(feat_ref, sp0_ref, sp1_ref, l1_ref, l2_ref, perm_ref, vr_ref,
             ws_ref, bs_ref, wo_ref, bo_ref, e1_ref, e2_ref, out_ref):
    feat = feat_ref[...]
    obj_iota = jax.lax.broadcasted_iota(jnp.int32, (N_REL, N_OBJ), 1)
    cls_iota = jax.lax.broadcasted_iota(jnp.int32, (N_REL, NUM_CLASSES), 1)
    rel_iota = jax.lax.broadcasted_iota(jnp.int32, (N_REL, N_REL), 1)

    oh_s = (obj_iota == sp0_ref[...]).astype(_BF)
    oh_o = (obj_iota == sp1_ref[...]).astype(_BF)
    g_s = jnp.dot(oh_s, _bd(feat), preferred_element_type=_F32)
    g_o = jnp.dot(oh_o, _bd(feat), preferred_element_type=_F32)
    subj = jnp.dot(_bd(g_s), _bd(ws_ref[...]),
                   preferred_element_type=_F32) + bs_ref[...]
    obj = jnp.dot(_bd(g_o), _bd(wo_ref[...]),
                  preferred_element_type=_F32) + bo_ref[...]

    oh_1 = (cls_iota == l1_ref[...]).astype(_F32)
    oh_2 = (cls_iota == l2_ref[...]).astype(_F32)
    emb1 = jnp.dot(oh_1, e1_ref[...], preferred_element_type=_F32)
    emb2 = jnp.dot(oh_2, e2_ref[...], preferred_element_type=_F32)

    oh_p = (rel_iota == perm_ref[...]).astype(_F32)
    vr_s = jnp.dot(oh_p, vr_ref[...], preferred_element_type=_F32)

    out_ref[...] = jnp.concatenate([subj, obj, vr_s, emb1, emb2], axis=1)


def _run_k0(features, sp0, sp1, l1, l2, perm, vr, ws, bs, wo, bo, e1, e2):
    full = lambda s: pl.BlockSpec(s, lambda: tuple(0 for _ in s))
    return pl.pallas_call(
        _k0_body,
        in_specs=[
            full((N_OBJ, IN_FEAT)),
            full((N_REL, 1)), full((N_REL, 1)),
            full((N_REL, 1)), full((N_REL, 1)), full((N_REL, 1)),
            full((N_REL, 512)),
            full((IN_FEAT, 512)), full((1, 512)),
            full((IN_FEAT, 512)), full((1, 512)),
            full((NUM_CLASSES, 200)), full((NUM_CLASSES, 200)),
        ],
        out_specs=full((N_REL, D_MODEL)),
        out_shape=jax.ShapeDtypeStruct((N_REL, D_MODEL), _F32),
    )(features, sp0, sp1, l1, l2, perm, vr, ws, bs, wo, bo, e1, e2)


def _k2_body(x_ref, w_ref, b_ref, out_ref):
    r = (jnp.dot(_bd(x_ref[...]), _bd(w_ref[...]), preferred_element_type=_F32)
         + b_ref[...]).astype(_BF)
    pad = jnp.zeros((N_REL, HEAD_PAD - HEAD_DIM), _BF)
    pieces = []
    for h in range(N_HEADS):
        pieces.append(r[:, h * HEAD_DIM:(h + 1) * HEAD_DIM])
        pieces.append(pad)
    out_ref[...] = jnp.concatenate(pieces, axis=1)


def _run_k2(x, w, b):
    full = lambda s: pl.BlockSpec(s, lambda: tuple(0 for _ in s))
    return pl.pallas_call(
        _k2_body,
        in_specs=[full((N_REL, D_MODEL)), full((D_MODEL, D_MODEL)),
                  full((1, D_MODEL))],
        out_specs=full((N_REL, QKV_PAD)),
        out_shape=jax.ShapeDtypeStruct((N_REL, QKV_PAD), _BF),
    )(x, w, b)


def _k3_body(q_ref, k_ref, v_ref, fc_ref, fr_ref, out_ref):
    q = q_ref[...]
    k = k_ref[...]
    scores = jax.lax.dot_general(q, k, (((1,), (1,)), ((), ())),
                                 preferred_element_type=_F32)
    scores = scores * np.float32(1.0 / np.sqrt(HEAD_DIM))
    mask = fc_ref[...] == fr_ref[...]
    scores = jnp.where(mask, scores, -1e9)
    m = jnp.max(scores, axis=1, keepdims=True)
    e = jnp.exp(scores - m)
    p = e / jnp.sum(e, axis=1, keepdims=True)
    out_ref[...] = _bd(jnp.dot(_bd(p), v_ref[...], preferred_element_type=_F32))


def _run_k3(q, k, v, fcol, frow):
    return pl.pallas_call(
        _k3_body,
        grid=(N_HEADS,),
        in_specs=[
            pl.BlockSpec((N_REL, HEAD_PAD), lambda h: (0, h)),
            pl.BlockSpec((N_REL, HEAD_PAD), lambda h: (0, h)),
            pl.BlockSpec((N_REL, HEAD_PAD), lambda h: (0, h)),
            pl.BlockSpec((N_REL, 1), lambda h: (0, 0)),
            pl.BlockSpec((1, N_REL), lambda h: (0, 0)),
        ],
        out_specs=pl.BlockSpec((N_REL, HEAD_PAD), lambda h: (0, h)),
        out_shape=jax.ShapeDtypeStruct((N_REL, QKV_PAD), _BF),
    )(q, k, v, fcol, frow)


def _layer_norm(y, g, b):
    n = np.float32(D_MODEL)
    mean = jnp.sum(y, axis=1, keepdims=True) / n
    var = jnp.sum(y * y, axis=1, keepdims=True) / n - mean * mean
    return (y - mean) * jax.lax.rsqrt(var + np.float32(1e-5)) * g + b


def _k4a_body(o_ref, w_ref, b_ref, x_ref, g_ref, bb_ref, out_ref):
    o = o_ref[...]
    o_c = jnp.concatenate(
        [o[:, h * HEAD_PAD:h * HEAD_PAD + HEAD_DIM] for h in range(N_HEADS)],
        axis=1)
    y = (jnp.dot(o_c, _bd(w_ref[...]), preferred_element_type=_F32)
         + b_ref[...] + x_ref[...])
    out_ref[...] = _layer_norm(y, g_ref[...], bb_ref[...])


def _run_k4a(o, wo, bo, x, g1, b1):
    full = lambda s: pl.BlockSpec(s, lambda: tuple(0 for _ in s))
    return pl.pallas_call(
        _k4a_body,
        in_specs=[full((N_REL, QKV_PAD)), full((D_MODEL, D_MODEL)),
                  full((1, D_MODEL)), full((N_REL, D_MODEL)),
                  full((1, D_MODEL)), full((1, D_MODEL))],
        out_specs=full((N_REL, D_MODEL)),
        out_shape=jax.ShapeDtypeStruct((N_REL, D_MODEL), _F32),
    )(o, wo, bo, x, g1, b1)


def _k4b_body(x_ref, w1_ref, b1_ref, w2_ref, b2_ref, g_ref, bb_ref, out_ref,
              *, nsteps):
    j = pl.program_id(0)
    h = jnp.maximum(jnp.dot(_bd(x_ref[...]), _bd(w1_ref[...]),
                            preferred_element_type=_F32) + b1_ref[...], 0.0)
    part = jnp.dot(_bd(h), _bd(w2_ref[...]), preferred_element_type=_F32)

    @pl.when(j == 0)
    def _():
        out_ref[...] = part

    @pl.when(j > 0)
    def _():
        out_ref[...] += part

    @pl.when(j == nsteps - 1)
    def _():
        y = out_ref[...] + b2_ref[...] + x_ref[...]
        out_ref[...] = _layer_norm(y, g_ref[...], bb_ref[...])


def _run_k4b(x, w1, b1, w2, b2, g2, bb2, bh=512):
    nsteps = D_FF // bh
    return pl.pallas_call(
        functools.partial(_k4b_body, nsteps=nsteps),
        grid=(nsteps,),
        in_specs=[
            pl.BlockSpec((N_REL, D_MODEL), lambda j: (0, 0)),
            pl.BlockSpec((D_MODEL, bh), lambda j: (0, j)),
            pl.BlockSpec((1, bh), lambda j: (0, j)),
            pl.BlockSpec((bh, D_MODEL), lambda j: (j, 0)),
            pl.BlockSpec((1, D_MODEL), lambda j: (0, 0)),
            pl.BlockSpec((1, D_MODEL), lambda j: (0, 0)),
            pl.BlockSpec((1, D_MODEL), lambda j: (0, 0)),
        ],
        out_specs=pl.BlockSpec((N_REL, D_MODEL), lambda j: (0, 0)),
        out_shape=jax.ShapeDtypeStruct((N_REL, D_MODEL), _F32),
    )(x, w1, b1, w2, b2, g2, bb2)


def kernel(features, pair_idx, union_feat, spatial_masks, pred_labels, boxes,
           params):
    p = params
    pair_idx = pair_idx.astype(jnp.int32)

    frame = boxes[pair_idx[:, 1], 0].astype(jnp.int32)
    perm = jnp.argsort(frame, stable=True).astype(jnp.int32)
    fs = frame[perm]
    sp = pair_idx[perm]
    sp0 = sp[:, 0:1]
    sp1 = sp[:, 1:2]
    l1 = pred_labels[sp[:, 0]].astype(jnp.int32).reshape(N_REL, 1)
    l2 = pred_labels[sp[:, 1]].astype(jnp.int32).reshape(N_REL, 1)
    fcol = fs.reshape(N_REL, 1)
    frow = fs.reshape(1, N_REL)
    perm2 = perm.reshape(N_REL, 1)

    u_flat = union_feat.reshape(N_REL, C_U * HW)
    s_flat = spatial_masks.reshape(N_REL, 2 * HW)
    bias_um = (p['bu'] + p['bm']).reshape(1, D_MID)
    wvrf = p['W_vr'].reshape(D_MID, HW * 512).astype(_BF)
    b_vr = p['b_vr'].reshape(1, 512)
    row = lambda v: v.reshape(1, -1)

    m2, m2s, const = _run_k1p(p['Wu'], wvrf, p['Wm'], bias_um, b_vr)
    vr = _run_k1m(u_flat, s_flat, m2, m2s, const)
    x = _run_k0(features, sp0, sp1, l1, l2, perm2, vr,
                p['W_subj'], row(p['b_subj']), p['W_obj'], row(p['b_obj']),
                p['emb1'], p['emb2'])
    q = _run_k2(x, p['Wq'], row(p['bq']))
    k = _run_k2(x, p['Wk'], row(p['bk']))
    v = _run_k2(x, p['Wv'], row(p['bv']))
    o = _run_k3(q, k, v, fcol, frow)
    x1 = _run_k4a(o, p['Wo'], row(p['bo']), x, row(p['ln1_g']), row(p['ln1_b']))
    out = _run_k4b(x1, p['W1'], row(p['b1']), p['W2'], row(p['b2']),
                   row(p['ln2_g']), row(p['ln2_b']))
    return out


# speedup vs baseline: 1.0068x; 1.0068x over previous
import functools

import jax
import jax.numpy as jnp
import numpy as np
from jax import lax
from jax.experimental import pallas as pl
from jax.experimental.pallas import tpu as pltpu
from jax.experimental.pallas import tpu_sc as plsc

N_OBJ = 600
N_REL = 1024
IN_FEAT = 2048
D_MODEL = 1936
N_HEADS = 8
HEAD_DIM = 242
HEAD_PAD = 256
QKV_PAD = N_HEADS * HEAD_PAD
NUM_CLASSES = 37
D_FF = 2048
HW = 49
C_U = 1024
D_MID = 256

_F32 = jnp.float32
_BF = jnp.bfloat16


def _bd(t):
    return t.astype(_BF)


def _k1p_body(wu_ref, wvrf_ref, wm_ref, bias_ref, bvr_ref,
              m2_ref, m2s_ref, const_ref, *, bc):
    c = pl.program_id(0)
    wvrf = wvrf_ref[...]
    m2 = jnp.dot(_bd(wu_ref[...]), wvrf,
                 preferred_element_type=_F32)
    m2_ref[...] = _bd(m2).reshape(bc * HW, 512)

    @pl.when(c == 0)
    def _():
        m2s = jnp.dot(_bd(wm_ref[...]), wvrf,
                      preferred_element_type=_F32)
        m2s_ref[...] = _bd(m2s).reshape(2 * HW, 512)
        cb = jnp.dot(_bd(bias_ref[...]), wvrf,
                     preferred_element_type=_F32)
        const_ref[...] = (jnp.sum(cb.reshape(HW, 512), axis=0, keepdims=True)
                          + bvr_ref[...])


def _run_k1p(wu, wvrf, wm, bias_um, b_vr, bc=128):
    grid = C_U // bc
    return pl.pallas_call(
        functools.partial(_k1p_body, bc=bc),
        grid=(grid,),
        in_specs=[
            pl.BlockSpec((bc, D_MID), lambda c: (c, 0)),
            pl.BlockSpec((D_MID, HW * 512), lambda c: (0, 0)),
            pl.BlockSpec((2, D_MID), lambda c: (0, 0)),
            pl.BlockSpec((1, D_MID), lambda c: (0, 0)),
            pl.BlockSpec((1, 512), lambda c: (0, 0)),
        ],
        out_specs=[
            pl.BlockSpec((bc * HW, 512), lambda c: (c, 0)),
            pl.BlockSpec((2 * HW, 512), lambda c: (0, 0)),
            pl.BlockSpec((1, 512), lambda c: (0, 0)),
        ],
        out_shape=[
            jax.ShapeDtypeStruct((C_U * HW, 512), _BF),
            jax.ShapeDtypeStruct((2 * HW, 512), _BF),
            jax.ShapeDtypeStruct((1, 512), _F32),
        ],
    )(wu, wvrf, wm, bias_um, b_vr)


def _k1m_body(u_ref, s_ref, m2_ref, m2s_ref, const_ref, out_ref):
    c = pl.program_id(1)
    part = jnp.dot(_bd(u_ref[...]), m2_ref[...],
                   preferred_element_type=_F32)

    @pl.when(c == 0)
    def _():
        out_ref[...] = (part + const_ref[...]
                        + jnp.dot(_bd(s_ref[...]), m2s_ref[...],
                                  preferred_element_type=_F32))

    @pl.when(c > 0)
    def _():
        out_ref[...] += part


def _run_k1m(u_flat, s_flat, m2, m2s, const, bn=512, bc=128):
    kb = bc * HW
    grid = (N_REL // bn, C_U // bc)
    return pl.pallas_call(
        _k1m_body,
        grid=grid,
        in_specs=[
            pl.BlockSpec((bn, kb), lambda n, c: (n, c)),
            pl.BlockSpec((bn, 2 * HW), lambda n, c: (n, 0)),
            pl.BlockSpec((kb, 512), lambda n, c: (c, 0)),
            pl.BlockSpec((2 * HW, 512), lambda n, c: (0, 0)),
            pl.BlockSpec((1, 512), lambda n, c: (0, 0)),
        ],
        out_specs=pl.BlockSpec((bn, 512), lambda n, c: (n, 0)),
        out_shape=jax.ShapeDtypeStruct((N_REL, 512), _F32),
    )(u_flat, s_flat, m2, m2s, const)


_SC_NC = 2
_SC_NS = 16
_SC_NW = _SC_NC * _SC_NS
_SC_B = 2 * N_REL
_SC_BPW = _SC_B // _SC_NW
_SC_CH = 16


def _run_sc_gather(features, idx2):
    mesh = plsc.VectorSubcoreMesh(core_axis_name="c", subcore_axis_name="s")

    @functools.partial(
        pl.kernel, mesh=mesh,
        out_type=jax.ShapeDtypeStruct((_SC_B, IN_FEAT), _F32),
        scratch_types=[
            pltpu.VMEM((_SC_CH,), jnp.int32),
            pltpu.VMEM((_SC_CH, IN_FEAT), _F32),
            pltpu.SemaphoreType.DMA,
        ],
    )
    def k(feat_hbm, idx_hbm, out_hbm, idx_v, rows_v, sem):
        wid = lax.axis_index("s") * _SC_NC + lax.axis_index("c")
        base = wid * _SC_BPW
        for ch in range(_SC_BPW // _SC_CH):
            off = base + ch * _SC_CH
            pltpu.sync_copy(idx_hbm.at[pl.ds(off, _SC_CH)], idx_v)
            pltpu.async_copy(feat_hbm.at[idx_v], rows_v, sem).wait()
            pltpu.sync_copy(rows_v, out_hbm.at[pl.ds(off, _SC_CH)])

    return k(features, idx2)


def _k0_body(g_ref, l1_ref, l2_ref, perm_ref, vr_ref,
             ws_ref, bs_ref, wo_ref, bo_ref, e1_ref, e2_ref, out_ref):
    cls_iota = jax.lax.broadcasted_iota(jnp.int32, (N_REL, NUM_CLASSES), 1)
    rel_iota = jax.lax.broadcasted_iota(jnp.int32, (N_REL, N_REL), 1)

    g = _bd(g_ref[...])
    subj = jnp.dot(g[:N_REL], _bd(ws_ref[...]),
                   preferred_element_type=_F32) + bs_ref[...]
    obj = jnp.dot(g[N_REL:], _bd(wo_ref[...]),
                  preferred_element_type=_F32) + bo_ref[...]

    oh_1 = (cls_iota == l1_ref[...]).astype(_F32)
    oh_2 = (cls_iota == l2_ref[...]).astype(_F32)
    emb1 = jnp.dot(oh_1, e1_ref[...], preferred_element_type=_F32)
    emb2 = jnp.dot(oh_2, e2_ref[...], preferred_element_type=_F32)

    oh_p = (rel_iota == perm_ref[...]).astype(_F32)
    vr_s = jnp.dot(oh_p, vr_ref[...], preferred_element_type=_F32)

    out_ref[...] = jnp.concatenate([subj, obj, vr_s, emb1, emb2], axis=1)


def _run_k0(g, l1, l2, perm, vr, ws, bs, wo, bo, e1, e2):
    full = lambda s: pl.BlockSpec(s, lambda: tuple(0 for _ in s))
    return pl.pallas_call(
        _k0_body,
        in_specs=[
            full((_SC_B, IN_FEAT)),
            full((N_REL, 1)), full((N_REL, 1)), full((N_REL, 1)),
            full((N_REL, 512)),
            full((IN_FEAT, 512)), full((1, 512)),
            full((IN_FEAT, 512)), full((1, 512)),
            full((NUM_CLASSES, 200)), full((NUM_CLASSES, 200)),
        ],
        out_specs=full((N_REL, D_MODEL)),
        out_shape=jax.ShapeDtypeStruct((N_REL, D_MODEL), _F32),
    )(g, l1, l2, perm, vr, ws, bs, wo, bo, e1, e2)


def _k2_body(x_ref, w_ref, b_ref, out_ref):
    r = (jnp.dot(_bd(x_ref[...]), _bd(w_ref[...]), preferred_element_type=_F32)
         + b_ref[...]).astype(_BF)
    pad = jnp.zeros((N_REL, HEAD_PAD - HEAD_DIM), _BF)
    pieces = []
    for h in range(N_HEADS):
        pieces.append(r[:, h * HEAD_DIM:(h + 1) * HEAD_DIM])
        pieces.append(pad)
    out_ref[...] = jnp.concatenate(pieces, axis=1)


def _run_k2(x, w, b):
    full = lambda s: pl.BlockSpec(s, lambda: tuple(0 for _ in s))
    return pl.pallas_call(
        _k2_body,
        in_specs=[full((N_REL, D_MODEL)), full((D_MODEL, D_MODEL)),
                  full((1, D_MODEL))],
        out_specs=full((N_REL, QKV_PAD)),
        out_shape=jax.ShapeDtypeStruct((N_REL, QKV_PAD), _BF),
    )(x, w, b)


def _k3_body(q_ref, k_ref, v_ref, fc_ref, fr_ref, out_ref):
    q = q_ref[...]
    k = k_ref[...]
    scores = jax.lax.dot_general(q, k, (((1,), (1,)), ((), ())),
                                 preferred_element_type=_F32)
    scores = scores * np.float32(1.0 / np.sqrt(HEAD_DIM))
    mask = fc_ref[...] == fr_ref[...]
    scores = jnp.where(mask, scores, -1e9)
    m = jnp.max(scores, axis=1, keepdims=True)
    e = jnp.exp(scores - m)
    p = e / jnp.sum(e, axis=1, keepdims=True)
    out_ref[...] = _bd(jnp.dot(_bd(p), v_ref[...], preferred_element_type=_F32))


def _run_k3(q, k, v, fcol, frow):
    return pl.pallas_call(
        _k3_body,
        grid=(N_HEADS,),
        in_specs=[
            pl.BlockSpec((N_REL, HEAD_PAD), lambda h: (0, h)),
            pl.BlockSpec((N_REL, HEAD_PAD), lambda h: (0, h)),
            pl.BlockSpec((N_REL, HEAD_PAD), lambda h: (0, h)),
            pl.BlockSpec((N_REL, 1), lambda h: (0, 0)),
            pl.BlockSpec((1, N_REL), lambda h: (0, 0)),
        ],
        out_specs=pl.BlockSpec((N_REL, HEAD_PAD), lambda h: (0, h)),
        out_shape=jax.ShapeDtypeStruct((N_REL, QKV_PAD), _BF),
    )(q, k, v, fcol, frow)


def _layer_norm(y, g, b):
    n = np.float32(D_MODEL)
    mean = jnp.sum(y, axis=1, keepdims=True) / n
    var = jnp.sum(y * y, axis=1, keepdims=True) / n - mean * mean
    return (y - mean) * jax.lax.rsqrt(var + np.float32(1e-5)) * g + b


def _k4a_body(o_ref, w_ref, b_ref, x_ref, g_ref, bb_ref, out_ref):
    o = o_ref[...]
    o_c = jnp.concatenate(
        [o[:, h * HEAD_PAD:h * HEAD_PAD + HEAD_DIM] for h in range(N_HEADS)],
        axis=1)
    y = (jnp.dot(o_c, _bd(w_ref[...]), preferred_element_type=_F32)
         + b_ref[...] + x_ref[...])
    out_ref[...] = _layer_norm(y, g_ref[...], bb_ref[...])


def _run_k4a(o, wo, bo, x, g1, b1):
    full = lambda s: pl.BlockSpec(s, lambda: tuple(0 for _ in s))
    return pl.pallas_call(
        _k4a_body,
        in_specs=[full((N_REL, QKV_PAD)), full((D_MODEL, D_MODEL)),
                  full((1, D_MODEL)), full((N_REL, D_MODEL)),
                  full((1, D_MODEL)), full((1, D_MODEL))],
        out_specs=full((N_REL, D_MODEL)),
        out_shape=jax.ShapeDtypeStruct((N_REL, D_MODEL), _F32),
    )(o, wo, bo, x, g1, b1)


def _k4b_body(x_ref, w1_ref, b1_ref, w2_ref, b2_ref, g_ref, bb_ref, out_ref,
              *, nsteps):
    j = pl.program_id(0)
    h = jnp.maximum(jnp.dot(_bd(x_ref[...]), _bd(w1_ref[...]),
                            preferred_element_type=_F32) + b1_ref[...], 0.0)
    part = jnp.dot(_bd(h), _bd(w2_ref[...]), preferred_element_type=_F32)

    @pl.when(j == 0)
    def _():
        out_ref[...] = part

    @pl.when(j > 0)
    def _():
        out_ref[...] += part

    @pl.when(j == nsteps - 1)
    def _():
        y = out_ref[...] + b2_ref[...] + x_ref[...]
        out_ref[...] = _layer_norm(y, g_ref[...], bb_ref[...])


def _run_k4b(x, w1, b1, w2, b2, g2, bb2, bh=512):
    nsteps = D_FF // bh
    return pl.pallas_call(
        functools.partial(_k4b_body, nsteps=nsteps),
        grid=(nsteps,),
        in_specs=[
            pl.BlockSpec((N_REL, D_MODEL), lambda j: (0, 0)),
            pl.BlockSpec((D_MODEL, bh), lambda j: (0, j)),
            pl.BlockSpec((1, bh), lambda j: (0, j)),
            pl.BlockSpec((bh, D_MODEL), lambda j: (j, 0)),
            pl.BlockSpec((1, D_MODEL), lambda j: (0, 0)),
            pl.BlockSpec((1, D_MODEL), lambda j: (0, 0)),
            pl.BlockSpec((1, D_MODEL), lambda j: (0, 0)),
        ],
        out_specs=pl.BlockSpec((N_REL, D_MODEL), lambda j: (0, 0)),
        out_shape=jax.ShapeDtypeStruct((N_REL, D_MODEL), _F32),
    )(x, w1, b1, w2, b2, g2, bb2)


def kernel(features, pair_idx, union_feat, spatial_masks, pred_labels, boxes,
           params):
    p = params
    pair_idx = pair_idx.astype(jnp.int32)

    frame = boxes[pair_idx[:, 1], 0].astype(jnp.int32)
    perm = jnp.argsort(frame, stable=True).astype(jnp.int32)
    fs = frame[perm]
    sp = pair_idx[perm]
    sp0 = sp[:, 0:1]
    sp1 = sp[:, 1:2]
    l1 = pred_labels[sp[:, 0]].astype(jnp.int32).reshape(N_REL, 1)
    l2 = pred_labels[sp[:, 1]].astype(jnp.int32).reshape(N_REL, 1)
    fcol = fs.reshape(N_REL, 1)
    frow = fs.reshape(1, N_REL)
    perm2 = perm.reshape(N_REL, 1)

    u_flat = union_feat.reshape(N_REL, C_U * HW)
    s_flat = spatial_masks.reshape(N_REL, 2 * HW)
    bias_um = (p['bu'] + p['bm']).reshape(1, D_MID)
    wvrf = p['W_vr'].reshape(D_MID, HW * 512).astype(_BF)
    b_vr = p['b_vr'].reshape(1, 512)
    row = lambda v: v.reshape(1, -1)

    idx2 = jnp.concatenate([sp[:, 0], sp[:, 1]]).astype(jnp.int32)
    g = _run_sc_gather(features, idx2)
    m2, m2s, const = _run_k1p(p['Wu'], wvrf, p['Wm'], bias_um, b_vr)
    vr = _run_k1m(u_flat, s_flat, m2, m2s, const)
    x = _run_k0(g, l1, l2, perm2, vr,
                p['W_subj'], row(p['b_subj']), p['W_obj'], row(p['b_obj']),
                p['emb1'], p['emb2'])
    q = _run_k2(x, p['Wq'], row(p['bq']))
    k = _run_k2(x, p['Wk'], row(p['bk']))
    v = _run_k2(x, p['Wv'], row(p['bv']))
    o = _run_k3(q, k, v, fcol, frow)
    x1 = _run_k4a(o, p['Wo'], row(p['bo']), x, row(p['ln1_g']), row(p['ln1_b']))
    out = _run_k4b(x1, p['W1'], row(p['b1']), p['W2'], row(p['b2']),
                   row(p['ln2_g']), row(p['ln2_b']))
    return out


# speedup vs baseline: 1.0117x; 1.0048x over previous
import functools

import jax
import jax.numpy as jnp
import numpy as np
from jax import lax
from jax.experimental import pallas as pl
from jax.experimental.pallas import tpu as pltpu
from jax.experimental.pallas import tpu_sc as plsc

N_OBJ = 600
N_REL = 1024
IN_FEAT = 2048
D_MODEL = 1936
N_HEADS = 8
HEAD_DIM = 242
HEAD_PAD = 256
QKV_PAD = N_HEADS * HEAD_PAD
NUM_CLASSES = 37
D_FF = 2048
HW = 49
C_U = 1024
D_MID = 256

_F32 = jnp.float32
_BF = jnp.bfloat16


def _bd(t):
    return t.astype(_BF)


def _k1p_body(wu_ref, wvrf_ref, wm_ref, bias_ref, bvr_ref,
              m2_ref, m2s_ref, const_ref, *, bc):
    c = pl.program_id(0)
    wvrf = wvrf_ref[...]
    m2 = jnp.dot(_bd(wu_ref[...]), wvrf,
                 preferred_element_type=_F32)
    m2_ref[...] = _bd(m2).reshape(bc * HW, 512)

    @pl.when(c == 0)
    def _():
        m2s = jnp.dot(_bd(wm_ref[...]), wvrf,
                      preferred_element_type=_F32)
        m2s_ref[...] = _bd(m2s).reshape(2 * HW, 512)
        cb = jnp.dot(_bd(bias_ref[...]), wvrf,
                     preferred_element_type=_F32)
        const_ref[...] = (jnp.sum(cb.reshape(HW, 512), axis=0, keepdims=True)
                          + bvr_ref[...])


def _run_k1p(wu, wvrf, wm, bias_um, b_vr, bc=128):
    grid = C_U // bc
    return pl.pallas_call(
        functools.partial(_k1p_body, bc=bc),
        grid=(grid,),
        in_specs=[
            pl.BlockSpec((bc, D_MID), lambda c: (c, 0)),
            pl.BlockSpec((D_MID, HW * 512), lambda c: (0, 0)),
            pl.BlockSpec((2, D_MID), lambda c: (0, 0)),
            pl.BlockSpec((1, D_MID), lambda c: (0, 0)),
            pl.BlockSpec((1, 512), lambda c: (0, 0)),
        ],
        out_specs=[
            pl.BlockSpec((bc * HW, 512), lambda c: (c, 0)),
            pl.BlockSpec((2 * HW, 512), lambda c: (0, 0)),
            pl.BlockSpec((1, 512), lambda c: (0, 0)),
        ],
        out_shape=[
            jax.ShapeDtypeStruct((C_U * HW, 512), _BF),
            jax.ShapeDtypeStruct((2 * HW, 512), _BF),
            jax.ShapeDtypeStruct((1, 512), _F32),
        ],
    )(wu, wvrf, wm, bias_um, b_vr)


def _k1m_body(u_ref, s_ref, m2_ref, m2s_ref, const_ref, out_ref):
    c = pl.program_id(1)
    part = jnp.dot(_bd(u_ref[...]), m2_ref[...],
                   preferred_element_type=_F32)

    @pl.when(c == 0)
    def _():
        out_ref[...] = (part + const_ref[...]
                        + jnp.dot(_bd(s_ref[...]), m2s_ref[...],
                                  preferred_element_type=_F32))

    @pl.when(c > 0)
    def _():
        out_ref[...] += part


def _run_k1m(u_flat, s_flat, m2, m2s, const, bn=512, bc=128):
    kb = bc * HW
    grid = (N_REL // bn, C_U // bc)
    return pl.pallas_call(
        _k1m_body,
        grid=grid,
        in_specs=[
            pl.BlockSpec((bn, kb), lambda n, c: (n, c)),
            pl.BlockSpec((bn, 2 * HW), lambda n, c: (n, 0)),
            pl.BlockSpec((kb, 512), lambda n, c: (c, 0)),
            pl.BlockSpec((2 * HW, 512), lambda n, c: (0, 0)),
            pl.BlockSpec((1, 512), lambda n, c: (0, 0)),
        ],
        out_specs=pl.BlockSpec((bn, 512), lambda n, c: (n, 0)),
        out_shape=jax.ShapeDtypeStruct((N_REL, 512), _F32),
    )(u_flat, s_flat, m2, m2s, const)


_SC_NC = 2
_SC_NS = 16
_SC_NW = _SC_NC * _SC_NS
_SC_B = 2 * N_REL
_SC_BPW = _SC_B // _SC_NW
_SC_CH = 16


def _run_sc_gather(features, idx2):
    mesh = plsc.VectorSubcoreMesh(core_axis_name="c", subcore_axis_name="s")

    @functools.partial(
        pl.kernel, mesh=mesh,
        out_type=jax.ShapeDtypeStruct((_SC_B, IN_FEAT), _F32),
        scratch_types=[
            pltpu.VMEM((_SC_CH,), jnp.int32),
            pltpu.VMEM((_SC_CH, IN_FEAT), _F32),
            pltpu.SemaphoreType.DMA,
        ],
    )
    def k(feat_hbm, idx_hbm, out_hbm, idx_v, rows_v, sem):
        wid = lax.axis_index("s") * _SC_NC + lax.axis_index("c")
        base = wid * _SC_BPW
        for ch in range(_SC_BPW // _SC_CH):
            off = base + ch * _SC_CH
            pltpu.sync_copy(idx_hbm.at[pl.ds(off, _SC_CH)], idx_v)
            pltpu.async_copy(feat_hbm.at[idx_v], rows_v, sem).wait()
            pltpu.sync_copy(rows_v, out_hbm.at[pl.ds(off, _SC_CH)])

    return k(features, idx2)


def _k0_body(g_ref, l1_ref, l2_ref, perm_ref, vr_ref,
             ws_ref, bs_ref, wo_ref, bo_ref, e1_ref, e2_ref, out_ref,
             outb_ref):
    cls_iota = jax.lax.broadcasted_iota(jnp.int32, (N_REL, NUM_CLASSES), 1)
    rel_iota = jax.lax.broadcasted_iota(jnp.int32, (N_REL, N_REL), 1)

    g = _bd(g_ref[...])
    subj = jnp.dot(g[:N_REL], _bd(ws_ref[...]),
                   preferred_element_type=_F32) + bs_ref[...]
    obj = jnp.dot(g[N_REL:], _bd(wo_ref[...]),
                  preferred_element_type=_F32) + bo_ref[...]

    oh_1 = (cls_iota == l1_ref[...]).astype(_F32)
    oh_2 = (cls_iota == l2_ref[...]).astype(_F32)
    emb1 = jnp.dot(oh_1, e1_ref[...], preferred_element_type=_F32)
    emb2 = jnp.dot(oh_2, e2_ref[...], preferred_element_type=_F32)

    oh_p = (rel_iota == perm_ref[...]).astype(_F32)
    vr_s = jnp.dot(oh_p, vr_ref[...], preferred_element_type=_F32)

    x = jnp.concatenate([subj, obj, vr_s, emb1, emb2], axis=1)
    out_ref[...] = x
    outb_ref[...] = _bd(x)


def _run_k0(g, l1, l2, perm, vr, ws, bs, wo, bo, e1, e2):
    full = lambda s: pl.BlockSpec(s, lambda: tuple(0 for _ in s))
    return pl.pallas_call(
        _k0_body,
        in_specs=[
            full((_SC_B, IN_FEAT)),
            full((N_REL, 1)), full((N_REL, 1)), full((N_REL, 1)),
            full((N_REL, 512)),
            full((IN_FEAT, 512)), full((1, 512)),
            full((IN_FEAT, 512)), full((1, 512)),
            full((NUM_CLASSES, 200)), full((NUM_CLASSES, 200)),
        ],
        out_specs=[full((N_REL, D_MODEL)), full((N_REL, D_MODEL))],
        out_shape=[jax.ShapeDtypeStruct((N_REL, D_MODEL), _F32),
                   jax.ShapeDtypeStruct((N_REL, D_MODEL), _BF)],
    )(g, l1, l2, perm, vr, ws, bs, wo, bo, e1, e2)


def _k2_body(x_ref, w_ref, b_ref, out_ref):
    r = (jnp.dot(x_ref[...], _bd(w_ref[...]), preferred_element_type=_F32)
         + b_ref[...]).astype(_BF)
    pad = jnp.zeros((N_REL, HEAD_PAD - HEAD_DIM), _BF)
    pieces = []
    for h in range(N_HEADS):
        pieces.append(r[:, h * HEAD_DIM:(h + 1) * HEAD_DIM])
        pieces.append(pad)
    out_ref[...] = jnp.concatenate(pieces, axis=1)


def _run_k2(x, w, b):
    full = lambda s: pl.BlockSpec(s, lambda: tuple(0 for _ in s))
    return pl.pallas_call(
        _k2_body,
        in_specs=[full((N_REL, D_MODEL)), full((D_MODEL, D_MODEL)),
                  full((1, D_MODEL))],
        out_specs=full((N_REL, QKV_PAD)),
        out_shape=jax.ShapeDtypeStruct((N_REL, QKV_PAD), _BF),
    )(x, w, b)


def _k3_body(q_ref, k_ref, v_ref, fc_ref, fr_ref, out_ref):
    q = q_ref[...]
    k = k_ref[...]
    scores = jax.lax.dot_general(q, k, (((1,), (1,)), ((), ())),
                                 preferred_element_type=_F32)
    scores = scores * np.float32(1.0 / np.sqrt(HEAD_DIM))
    mask = fc_ref[...] == fr_ref[...]
    scores = jnp.where(mask, scores, -1e9)
    m = jnp.max(scores, axis=1, keepdims=True)
    e = jnp.exp(scores - m)
    p = e / jnp.sum(e, axis=1, keepdims=True)
    out_ref[...] = _bd(jnp.dot(_bd(p), v_ref[...], preferred_element_type=_F32))


def _run_k3(q, k, v, fcol, frow):
    return pl.pallas_call(
        _k3_body,
        grid=(N_HEADS,),
        in_specs=[
            pl.BlockSpec((N_REL, HEAD_PAD), lambda h: (0, h)),
            pl.BlockSpec((N_REL, HEAD_PAD), lambda h: (0, h)),
            pl.BlockSpec((N_REL, HEAD_PAD), lambda h: (0, h)),
            pl.BlockSpec((N_REL, 1), lambda h: (0, 0)),
            pl.BlockSpec((1, N_REL), lambda h: (0, 0)),
        ],
        out_specs=pl.BlockSpec((N_REL, HEAD_PAD), lambda h: (0, h)),
        out_shape=jax.ShapeDtypeStruct((N_REL, QKV_PAD), _BF),
    )(q, k, v, fcol, frow)


def _layer_norm(y, g, b):
    n = np.float32(D_MODEL)
    mean = jnp.sum(y, axis=1, keepdims=True) / n
    var = jnp.sum(y * y, axis=1, keepdims=True) / n - mean * mean
    return (y - mean) * jax.lax.rsqrt(var + np.float32(1e-5)) * g + b


def _k4a_body(o_ref, w_ref, b_ref, x_ref, g_ref, bb_ref, out_ref):
    o = o_ref[...]
    o_c = jnp.concatenate(
        [o[:, h * HEAD_PAD:h * HEAD_PAD + HEAD_DIM] for h in range(N_HEADS)],
        axis=1)
    y = (jnp.dot(o_c, _bd(w_ref[...]), preferred_element_type=_F32)
         + b_ref[...] + x_ref[...])
    out_ref[...] = _layer_norm(y, g_ref[...], bb_ref[...])


def _run_k4a(o, wo, bo, x, g1, b1):
    full = lambda s: pl.BlockSpec(s, lambda: tuple(0 for _ in s))
    return pl.pallas_call(
        _k4a_body,
        in_specs=[full((N_REL, QKV_PAD)), full((D_MODEL, D_MODEL)),
                  full((1, D_MODEL)), full((N_REL, D_MODEL)),
                  full((1, D_MODEL)), full((1, D_MODEL))],
        out_specs=full((N_REL, D_MODEL)),
        out_shape=jax.ShapeDtypeStruct((N_REL, D_MODEL), _F32),
    )(o, wo, bo, x, g1, b1)


def _k4b_body(x_ref, w1_ref, b1_ref, w2_ref, b2_ref, g_ref, bb_ref, out_ref,
              *, nsteps):
    j = pl.program_id(0)
    h = jnp.maximum(jnp.dot(_bd(x_ref[...]), _bd(w1_ref[...]),
                            preferred_element_type=_F32) + b1_ref[...], 0.0)
    part = jnp.dot(_bd(h), _bd(w2_ref[...]), preferred_element_type=_F32)

    @pl.when(j == 0)
    def _():
        out_ref[...] = part

    @pl.when(j > 0)
    def _():
        out_ref[...] += part

    @pl.when(j == nsteps - 1)
    def _():
        y = out_ref[...] + b2_ref[...] + x_ref[...]
        out_ref[...] = _layer_norm(y, g_ref[...], bb_ref[...])


def _run_k4b(x, w1, b1, w2, b2, g2, bb2, bh=512):
    nsteps = D_FF // bh
    return pl.pallas_call(
        functools.partial(_k4b_body, nsteps=nsteps),
        grid=(nsteps,),
        in_specs=[
            pl.BlockSpec((N_REL, D_MODEL), lambda j: (0, 0)),
            pl.BlockSpec((D_MODEL, bh), lambda j: (0, j)),
            pl.BlockSpec((1, bh), lambda j: (0, j)),
            pl.BlockSpec((bh, D_MODEL), lambda j: (j, 0)),
            pl.BlockSpec((1, D_MODEL), lambda j: (0, 0)),
            pl.BlockSpec((1, D_MODEL), lambda j: (0, 0)),
            pl.BlockSpec((1, D_MODEL), lambda j: (0, 0)),
        ],
        out_specs=pl.BlockSpec((N_REL, D_MODEL), lambda j: (0, 0)),
        out_shape=jax.ShapeDtypeStruct((N_REL, D_MODEL), _F32),
    )(x, w1, b1, w2, b2, g2, bb2)


def kernel(features, pair_idx, union_feat, spatial_masks, pred_labels, boxes,
           params):
    p = params
    pair_idx = pair_idx.astype(jnp.int32)

    frame = boxes[pair_idx[:, 1], 0].astype(jnp.int32)
    perm = jnp.argsort(frame, stable=True).astype(jnp.int32)
    fs = frame[perm]
    sp = pair_idx[perm]
    l1 = pred_labels[sp[:, 0]].astype(jnp.int32).reshape(N_REL, 1)
    l2 = pred_labels[sp[:, 1]].astype(jnp.int32).reshape(N_REL, 1)
    fcol = fs.reshape(N_REL, 1)
    frow = fs.reshape(1, N_REL)
    perm2 = perm.reshape(N_REL, 1)

    u_flat = union_feat.reshape(N_REL, C_U * HW)
    s_flat = spatial_masks.reshape(N_REL, 2 * HW)
    bias_um = (p['bu'] + p['bm']).reshape(1, D_MID)
    wvrf = p['W_vr'].reshape(D_MID, HW * 512).astype(_BF)
    b_vr = p['b_vr'].reshape(1, 512)
    row = lambda v: v.reshape(1, -1)

    idx2 = jnp.concatenate([sp[:, 0], sp[:, 1]]).astype(jnp.int32)
    g = _run_sc_gather(features, idx2)
    m2, m2s, const = _run_k1p(p['Wu'], wvrf, p['Wm'], bias_um, b_vr)
    vr = _run_k1m(u_flat, s_flat, m2, m2s, const)
    x, xb = _run_k0(g, l1, l2, perm2, vr,
                    p['W_subj'], row(p['b_subj']), p['W_obj'],
                    row(p['b_obj']), p['emb1'], p['emb2'])
    q = _run_k2(xb, p['Wq'], row(p['bq']))
    k = _run_k2(xb, p['Wk'], row(p['bk']))
    v = _run_k2(xb, p['Wv'], row(p['bv']))
    o = _run_k3(q, k, v, fcol, frow)
    x1 = _run_k4a(o, p['Wo'], row(p['bo']), x, row(p['ln1_g']), row(p['ln1_b']))
    out = _run_k4b(x1, p['W1'], row(p['b1']), p['W2'], row(p['b2']),
                   row(p['ln2_g']), row(p['ln2_b']))
    return out
